# baseline (speedup 1.0000x reference)
import functools
import math

import jax
import jax.numpy as jnp
from jax import lax
from jax.experimental import pallas as pl
from jax.experimental.pallas import tpu as pltpu
from jax.experimental.pallas import tpu_sc as plsc

VOCAB = 1000000
EMB_DIM = 64
BATCH = 16384
HIST = 50
B = BATCH * HIST
NC = 2
NS = 16
NW = NC * NS
BPW = B // NW
CHUNK = 128
NCHUNK = BPW // CHUNK
LANES = 16
SCALE = math.sqrt(EMB_DIM)


def _embed_body(x_hbm, table_hbm, out_hbm, idx_v, rows_v, gsem):
    cid = lax.axis_index("c")
    sid = lax.axis_index("s")
    wid = sid * NC + cid
    base = wid * BPW

    def chunk_step(g, carry):
        off = base + g * CHUNK
        pltpu.sync_copy(x_hbm.at[pl.ds(off, CHUNK)], idx_v)
        pltpu.async_copy(table_hbm.at[idx_v], rows_v, gsem).wait()

        def rowgrp(r4, c2):
            r0 = r4 * 4
            for dr in range(4):
                for q in range(4):
                    sl = (r0 + dr, pl.ds(q * LANES, LANES))
                    rows_v[sl] = rows_v[sl] * SCALE
            return c2

        lax.fori_loop(0, CHUNK // 4, rowgrp, 0)

        pltpu.sync_copy(rows_v, out_hbm.at[pl.ds(off, CHUNK)])
        return carry

    lax.fori_loop(0, NCHUNK, chunk_step, 0)


@functools.partial(
    pl.kernel,
    mesh=plsc.VectorSubcoreMesh(core_axis_name="c", subcore_axis_name="s"),
    out_type=jax.ShapeDtypeStruct((B, EMB_DIM), jnp.float32),
    scratch_types=[
        pltpu.VMEM((CHUNK,), jnp.int32),
        pltpu.VMEM((CHUNK, EMB_DIM), jnp.float32),
        pltpu.SemaphoreType.DMA,
    ],
    compiler_params=pltpu.CompilerParams(use_tc_tiling_on_sc=False),
)
def _embed_sc(x_hbm, table_hbm, out_hbm, idx_v, rows_v, gsem):
    _embed_body(x_hbm, table_hbm, out_hbm, idx_v, rows_v, gsem)


def kernel(x, table):
    x_flat = x.reshape(-1).astype(jnp.int32)
    out = _embed_sc(x_flat, table)
    return out.reshape(BATCH, HIST, EMB_DIM)


# speedup vs baseline: 1.2245x; 1.2245x over previous
import functools
import math

import jax
import jax.numpy as jnp
from jax import lax
from jax.experimental import pallas as pl
from jax.experimental.pallas import tpu as pltpu
from jax.experimental.pallas import tpu_sc as plsc

VOCAB = 1000000
EMB_DIM = 64
BATCH = 16384
HIST = 50
B = BATCH * HIST
NC = 2
NS = 16
NW = NC * NS
BPW = B // NW
G = 128
NG = 4
C = NG * G
NCHUNK = BPW // C
NB = 3
LANES = 16
SCALE = math.sqrt(EMB_DIM)


def _embed_body(x_hbm, table_hbm, out_hbm, idx_all, rows, gsem, osem):
    cid = lax.axis_index("c")
    sid = lax.axis_index("s")
    wid = sid * NC + cid
    base = wid * BPW

    pltpu.sync_copy(x_hbm.at[pl.ds(base, BPW)], idx_all)

    def fire_gathers(g, b):
        for j in range(NG):
            pltpu.async_copy(
                table_hbm.at[idx_all.at[pl.ds(g * C + j * G, G)]],
                rows[b].at[pl.ds(j * G, G)],
                gsem[b],
            )

    def drain_gathers(g, b):
        for j in range(NG):
            pltpu.make_async_copy(
                table_hbm.at[idx_all.at[pl.ds(g * C + j * G, G)]],
                rows[b].at[pl.ds(j * G, G)],
                gsem[b],
            ).wait()

    def fire_store(g, b):
        pltpu.async_copy(rows[b], out_hbm.at[pl.ds(base + g * C, C)], osem[b])

    def drain_store(b):
        pltpu.make_async_copy(rows[b], out_hbm.at[pl.ds(base, C)], osem[b]).wait()

    def scale(b):
        def rowgrp(r4, c2):
            r0 = r4 * 4
            for dr in range(4):
                for q in range(4):
                    sl = (r0 + dr, pl.ds(q * LANES, LANES))
                    rows[b][sl] = rows[b][sl] * SCALE
            return c2

        lax.fori_loop(0, C // 4, rowgrp, 0)

    def halfstep(g, b, fire_next):
        bn = (b + 1) % NB

        @pl.when(g >= 2)
        def _():
            drain_store(bn)

        if fire_next:
            fire_gathers(g + 1, bn)
        drain_gathers(g, b)
        scale(b)
        fire_store(g, b)

    fire_gathers(0, 0)

    nsup = (NCHUNK - 2) // NB

    def superstep(t, c2):
        for k in range(NB):
            halfstep(t * NB + k, k, fire_next=True)
        return c2

    lax.fori_loop(0, nsup, superstep, 0)

    halfstep(NCHUNK - 2, 0, fire_next=True)
    halfstep(NCHUNK - 1, 1, fire_next=False)
    drain_store(0)
    drain_store(1)


@functools.partial(
    pl.kernel,
    mesh=plsc.VectorSubcoreMesh(core_axis_name="c", subcore_axis_name="s"),
    out_type=jax.ShapeDtypeStruct((B, EMB_DIM), jnp.float32),
    scratch_types=[
        pltpu.VMEM((BPW,), jnp.int32),
        [pltpu.VMEM((C, EMB_DIM), jnp.float32) for _ in range(NB)],
        [pltpu.SemaphoreType.DMA for _ in range(NB)],
        [pltpu.SemaphoreType.DMA for _ in range(NB)],
    ],
    compiler_params=pltpu.CompilerParams(use_tc_tiling_on_sc=False),
)
def _embed_sc(x_hbm, table_hbm, out_hbm, idx_all, rows, gsem, osem):
    _embed_body(x_hbm, table_hbm, out_hbm, idx_all, rows, gsem, osem)


def kernel(x, table):
    x_flat = x.reshape(-1).astype(jnp.int32)
    out = _embed_sc(x_flat, table)
    return out.reshape(BATCH, HIST, EMB_DIM)
